# baseline (speedup 1.0000x reference)
import functools

import jax
import jax.numpy as jnp
from jax import lax
from jax.experimental import pallas as pl
from jax.experimental.pallas import tpu as pltpu
from jax.experimental.pallas import tpu_sc as plsc

VOCAB = 1000
EMBED_DIM = 128
BATCH = 4096
HIST = 200

_INFO = plsc.get_sparse_core_info()
NC = _INFO.num_cores
NS = _INFO.num_subcores
NW = NC * NS
TOTAL = BATCH * HIST
CHUNK = 128
PER_W = TOTAL // NW
NCHUNK = PER_W // CHUNK
NBUF = 5
PREF = 3

_mesh = plsc.VectorSubcoreMesh(core_axis_name="c", subcore_axis_name="s")


@functools.partial(
    pl.kernel,
    mesh=_mesh,
    out_type=jax.ShapeDtypeStruct((TOTAL, EMBED_DIM), jnp.float32),
    scratch_types=[
        pltpu.VMEM((NCHUNK, CHUNK), jnp.int32),
        pltpu.VMEM((NBUF, CHUNK, EMBED_DIM), jnp.float32),
        pltpu.VMEM_SHARED((VOCAB, EMBED_DIM), jnp.float32),
        pltpu.SemaphoreType.DMA((NBUF,)),
        pltpu.SemaphoreType.DMA((NBUF,)),
    ],
)
def _gather_kernel(idx_hbm, table_hbm, out_hbm, idx_v, rows_v, table_sh, gsem, ssem):
    sid = lax.axis_index("s")
    wid = sid * NC + lax.axis_index("c")
    base = wid * PER_W

    @pl.when(sid == 0)
    def _stage():
        pltpu.sync_copy(table_hbm, table_sh)

    pltpu.sync_copy(idx_hbm.at[wid], idx_v)
    plsc.subcore_barrier()

    def gather(j, b):
        src = table_hbm if b % 2 else table_sh
        return pltpu.make_async_copy(
            src.at[idx_v.at[j]], rows_v.at[b], gsem.at[b])

    def store(j, b):
        return pltpu.make_async_copy(
            rows_v.at[b], out_hbm.at[pl.ds(base + j * CHUNK, CHUNK)], ssem.at[b])

    for c in range(PREF):
        gather(c, c % NBUF).start()

    def step(i, carry):
        for u in range(NBUF):
            j = i * NBUF + u
            gather(j, u).wait()
            store(j, u).start()
            bn = (u + PREF) % NBUF
            if u < NBUF - PREF:
                @pl.when(i > 0)
                def _wait_prev():
                    store(j + PREF - NBUF, bn).wait()
                gather(j + PREF, bn).start()
            else:
                store(j + PREF - NBUF, bn).wait()

                @pl.when(j + PREF < NCHUNK)
                def _prefetch():
                    gather(j + PREF, bn).start()
        return carry

    lax.fori_loop(0, NCHUNK // NBUF, step, 0)

    for c in range(NCHUNK - (NBUF - PREF), NCHUNK):
        store(c, c % NBUF).wait()


def kernel(batch, weight):
    idx = batch.astype(jnp.int32).reshape(NW, NCHUNK, CHUNK)
    out = _gather_kernel(idx, weight)
    return out.reshape(BATCH, HIST, EMBED_DIM)


# speedup vs baseline: 1.5148x; 1.5148x over previous
import functools

import jax
import jax.numpy as jnp
from jax import lax
from jax.experimental import pallas as pl
from jax.experimental.pallas import tpu as pltpu
from jax.experimental.pallas import tpu_sc as plsc

VOCAB = 1000
EMBED_DIM = 128
BATCH = 4096
HIST = 200

_INFO = plsc.get_sparse_core_info()
NC = _INFO.num_cores
NS = _INFO.num_subcores
NW = NC * NS
TOTAL = BATCH * HIST
CHUNK = 128
PER_W = TOTAL // NW
NCHUNK = PER_W // CHUNK
NBUF = 5
PREF = 3

_mesh = plsc.VectorSubcoreMesh(core_axis_name="c", subcore_axis_name="s")


@functools.partial(
    pl.kernel,
    mesh=_mesh,
    out_type=jax.ShapeDtypeStruct((TOTAL, EMBED_DIM), jnp.float32),
    scratch_types=[
        pltpu.VMEM((NCHUNK, CHUNK), jnp.int32),
        pltpu.VMEM((NBUF, CHUNK, EMBED_DIM), jnp.float32),
        pltpu.VMEM_SHARED((VOCAB, EMBED_DIM), jnp.float32),
        pltpu.SemaphoreType.DMA((NBUF,)),
        pltpu.SemaphoreType.DMA((NBUF,)),
    ],
)
def _gather_kernel(idx_hbm, table_hbm, out_hbm, idx_v, rows_v, table_sh, gsem, ssem):
    sid = lax.axis_index("s")
    wid = sid * NC + lax.axis_index("c")
    base = wid * PER_W

    @pl.when(sid == 0)
    def _stage():
        pltpu.sync_copy(table_hbm, table_sh)

    pltpu.sync_copy(idx_hbm.at[wid], idx_v)
    plsc.subcore_barrier()

    def gather(j, b):
        return pltpu.make_async_copy(
            table_sh.at[idx_v.at[j]], rows_v.at[b], gsem.at[b])

    def store(j, b):
        return pltpu.make_async_copy(
            rows_v.at[b], out_hbm.at[pl.ds(base + j * CHUNK, CHUNK)], ssem.at[b])

    for c in range(PREF):
        gather(c, c % NBUF).start()

    def step(i, carry):
        for u in range(NBUF):
            j = i * NBUF + u
            gather(j, u).wait()
            store(j, u).start()
            bn = (u + PREF) % NBUF
            if u < NBUF - PREF:
                @pl.when(i > 0)
                def _wait_prev():
                    store(j + PREF - NBUF, bn).wait()
                gather(j + PREF, bn).start()
            else:
                store(j + PREF - NBUF, bn).wait()

                @pl.when(j + PREF < NCHUNK)
                def _prefetch():
                    gather(j + PREF, bn).start()
        return carry

    lax.fori_loop(0, NCHUNK // NBUF, step, 0)

    for c in range(NCHUNK - (NBUF - PREF), NCHUNK):
        store(c, c % NBUF).wait()


def kernel(batch, weight):
    idx = batch.astype(jnp.int32).reshape(NW, NCHUNK, CHUNK)
    out = _gather_kernel(idx, weight)
    return out.reshape(BATCH, HIST, EMBED_DIM)
